# Initial kernel scaffold; baseline (speedup 1.0000x reference)
import jax
import jax.numpy as jnp
from jax.experimental import pallas as pl


def kernel(x_node, edge_index_node_to_node, W1, a_src1, a_dst1, b1, W2, a_src2, a_dst2, b2, lin_W, lin_b):
    raise NotImplementedError("write your pallas kernel here")


# speedup vs baseline: 33.6381x; 33.6381x over previous
import functools

import jax
import jax.numpy as jnp
from jax import lax
from jax.experimental import pallas as pl
from jax.experimental.pallas import tpu as pltpu

_N = 1024
_E = 32768
_D = 128
_H = 2
_O1 = 256
_O2 = 64
_SLOPE = 0.2

_EC = 2048
_NB = 256


def _cbuild_body(src_ref, dst_ref, c_ref):
    ib = pl.program_id(0)
    jb = pl.program_id(1)
    ec = pl.program_id(2)

    @pl.when(ec == 0)
    def _():
        c_ref[...] = jnp.zeros_like(c_ref)

    sv = src_ref[...]
    dv = dst_ref[...]
    ji = jb * _NB + lax.broadcasted_iota(jnp.int32, (_EC, _NB), 1)
    ii = ib * _NB + lax.broadcasted_iota(jnp.int32, (_EC, _NB), 1)
    s_oh = (sv == ji).astype(jnp.bfloat16)
    d_oh = (dv == ii).astype(jnp.bfloat16)
    c_ref[...] += lax.dot_general(
        d_oh, s_oh, (((0,), (0,)), ((), ())),
        preferred_element_type=jnp.float32)


def _build_count_matrix(src, dst):
    return pl.pallas_call(
        _cbuild_body,
        grid=(_N // _NB, _N // _NB, _E // _EC),
        in_specs=[
            pl.BlockSpec((_EC, 1), lambda i, j, e: (e, 0)),
            pl.BlockSpec((_EC, 1), lambda i, j, e: (e, 0)),
        ],
        out_specs=pl.BlockSpec((_NB, _NB), lambda i, j, e: (i, j)),
        out_shape=jax.ShapeDtypeStruct((_N, _N), jnp.float32),
    )(src, dst)


def _proj_body(x_ref, w_ref, asrc_ref, adst_ref, xp_ref, ast_ref, adc_ref,
               *, heads, out_c):
    xp = jnp.dot(x_ref[...], w_ref[...], preferred_element_type=jnp.float32)
    xp_ref[...] = xp
    ast_rows = []
    adc_cols = []
    for h in range(heads):
        xph = xp[:, h * out_c:(h + 1) * out_c]
        a_s = asrc_ref[h:h + 1, :]
        a_d = adst_ref[h:h + 1, :]
        ast_rows.append(lax.dot_general(
            a_s, xph, (((1,), (1,)), ((), ())),
            preferred_element_type=jnp.float32))
        adc_cols.append(lax.dot_general(
            xph, a_d, (((1,), (1,)), ((), ())),
            preferred_element_type=jnp.float32))
    ast_ref[...] = jnp.concatenate(ast_rows, axis=0)
    adc_ref[...] = jnp.concatenate(adc_cols, axis=1)


def _project(x, w, a_src, a_dst, heads, out_c):
    n = x.shape[0]
    return pl.pallas_call(
        functools.partial(_proj_body, heads=heads, out_c=out_c),
        out_shape=[
            jax.ShapeDtypeStruct((n, heads * out_c), jnp.float32),
            jax.ShapeDtypeStruct((heads, n), jnp.float32),
            jax.ShapeDtypeStruct((n, heads), jnp.float32),
        ],
    )(x, w, a_src, a_dst)


_BI = 128


def _gat_body(c_ref, ast_ref, adc_ref, xp_ref, b_ref, out_ref,
              *, heads, out_c, relu):
    i = pl.program_id(0)
    cm = c_ref[...]
    parts = []
    for h in range(heads):
        as_row = ast_ref[h:h + 1, :]
        ad_col = adc_ref[pl.ds(i * _BI, _BI), h:h + 1]
        m = jnp.max(ast_ref[h, :]) + jnp.max(adc_ref[:, h])
        m = jnp.where(m >= 0, m, _SLOPE * m)
        raw = ad_col + as_row
        e = jnp.where(raw >= 0, raw, _SLOPE * raw)
        w = cm * jnp.exp(e - m)
        den = jnp.sum(w, axis=1, keepdims=True)
        alpha = w / (den + 1e-16)
        parts.append(jnp.dot(alpha, xp_ref[:, h * out_c:(h + 1) * out_c],
                             preferred_element_type=jnp.float32))
    out = jnp.concatenate(parts, axis=1) + b_ref[...]
    if relu:
        out = jnp.maximum(out, 0.0)
    out_ref[...] = out


def _gat_layer(c, ast, adc, xp, b, heads, out_c, relu):
    ho = heads * out_c
    return pl.pallas_call(
        functools.partial(_gat_body, heads=heads, out_c=out_c, relu=relu),
        grid=(_N // _BI,),
        in_specs=[
            pl.BlockSpec((_BI, _N), lambda i: (i, 0)),
            pl.BlockSpec((heads, _N), lambda i: (0, 0)),
            pl.BlockSpec((_N, heads), lambda i: (0, 0)),
            pl.BlockSpec((_N, ho), lambda i: (0, 0)),
            pl.BlockSpec((1, ho), lambda i: (0, 0)),
        ],
        out_specs=pl.BlockSpec((_BI, ho), lambda i: (i, 0)),
        out_shape=jax.ShapeDtypeStruct((_N, ho), jnp.float32),
    )(c, ast, adc, xp, b)


def _final_body(h_ref, lw_ref, lb_ref, out_ref):
    hm = h_ref[...]
    lw = lw_ref[...]
    s_col = jnp.dot(hm, lw, preferred_element_type=jnp.float32)
    s_row = lax.dot_general(lw, hm, (((0,), (1,)), ((), ())),
                            preferred_element_type=jnp.float32)
    out_ref[...] = s_col + s_row + lb_ref[0, 0]


def _final(h, lin_w, lin_b):
    return pl.pallas_call(
        _final_body,
        out_shape=jax.ShapeDtypeStruct((_N, _N), jnp.float32),
    )(h, lin_w, lin_b)


def kernel(x_node, edge_index_node_to_node, W1, a_src1, a_dst1, b1,
           W2, a_src2, a_dst2, b2, lin_W, lin_b):
    ei = edge_index_node_to_node.astype(jnp.int32)
    src = ei[0].reshape(_E, 1)
    dst = ei[1].reshape(_E, 1)

    c = _build_count_matrix(src, dst)

    xp1, ast1, adc1 = _project(x_node, W1, a_src1, a_dst1, _H, _O1)
    h1 = _gat_layer(c, ast1, adc1, xp1, b1.reshape(1, -1), _H, _O1, relu=True)

    xp2, ast2, adc2 = _project(h1, W2, a_src2, a_dst2, _H, _O2)
    h2 = _gat_layer(c, ast2, adc2, xp2, b2.reshape(1, -1), _H, _O2, relu=False)

    out = _final(h2, lin_W, lin_b.reshape(1, 1))
    return out.reshape(_N * _N, 1)
